# baseline (speedup 1.0000x reference)
import functools
import math

import jax
import jax.numpy as jnp
from jax import lax
from jax.experimental import pallas as pl
from jax.experimental.pallas import tpu as pltpu
from jax.experimental.pallas import tpu_sc as plsc

_NW = 32
_CHUNK = 400
_BS1 = 4000
_BS3 = 256


def _mlp_pack_body(t_ref, w1_ref, b1_ref, w2_ref, b2_ref, o_ref):
    t = t_ref[...]
    h = jnp.dot(t, w1_ref[...], preferred_element_type=jnp.float32) + b1_ref[...]
    h = 0.5 * h * (1.0 + lax.erf(h * (1.0 / math.sqrt(2.0))))
    h = jnp.dot(h, w2_ref[...], preferred_element_type=jnp.float32) + b2_ref[...]
    o_ref[...] = jnp.concatenate([h, h], axis=1)


def _mlp_pack(table, W1, b1, W2, b2):
    V, D = table.shape
    return pl.pallas_call(
        _mlp_pack_body,
        grid=(V // _BS1,),
        in_specs=[
            pl.BlockSpec((_BS1, D), lambda i: (i, 0)),
            pl.BlockSpec((D, D), lambda i: (0, 0)),
            pl.BlockSpec((1, D), lambda i: (0, 0)),
            pl.BlockSpec((D, D), lambda i: (0, 0)),
            pl.BlockSpec((1, D), lambda i: (0, 0)),
        ],
        out_specs=pl.BlockSpec((_BS1, 2 * D), lambda i: (i, 0)),
        out_shape=jax.ShapeDtypeStruct((V, 2 * D), jnp.float32),
    )(table, W1, b1, W2, b2)


def _make_sc_gather(D2, NIDX):
    b_per_w = NIDX // _NW
    n_chunks = b_per_w // _CHUNK
    n_super = n_chunks // 2
    mesh = plsc.VectorSubcoreMesh(core_axis_name="c", subcore_axis_name="s")

    n_quad = n_chunks // 4

    @functools.partial(
        pl.kernel,
        mesh=mesh,
        out_type=jax.ShapeDtypeStruct((NIDX, D2), jnp.float32),
        scratch_types=[
            [pltpu.VMEM((_CHUNK,), jnp.int32) for _ in range(2)],
            [pltpu.VMEM((_CHUNK,), jnp.int32) for _ in range(4)],
            [pltpu.VMEM((_CHUNK, D2), jnp.float32) for _ in range(2)],
            [pltpu.SemaphoreType.DMA for _ in range(4)],
            [pltpu.SemaphoreType.DMA for _ in range(4)],
            [pltpu.SemaphoreType.DMA for _ in range(2)],
            [pltpu.SemaphoreType.DMA for _ in range(2)],
        ],
    )
    def gather_k(idx_hbm, dst_hbm, tab_hbm, out_hbm,
                 idx_vs, dst_vs, rows_vs, lisems, ldsems, gsems, ssems):
        wid = lax.axis_index("s") * 2 + lax.axis_index("c")
        wbase = wid * b_per_w

        def start_load(i, b2, b4):
            base = wbase + i * _CHUNK
            pltpu.async_copy(idx_hbm.at[pl.ds(base, _CHUNK)],
                             idx_vs[b2], lisems[b4])
            pltpu.async_copy(dst_hbm.at[pl.ds(base, _CHUNK)],
                             dst_vs[b4], ldsems[b4])

        def wait_load(b2, b4):
            pltpu.make_async_copy(idx_hbm.at[pl.ds(wbase, _CHUNK)],
                                  idx_vs[b2], lisems[b4]).wait()
            pltpu.make_async_copy(dst_hbm.at[pl.ds(wbase, _CHUNK)],
                                  dst_vs[b4], ldsems[b4]).wait()

        def scatter_copy(b2, b4):
            return pltpu.make_async_copy(rows_vs[b2], out_hbm.at[dst_vs[b4]],
                                         ssems[b2])

        start_load(0, 0, 0)
        start_load(1, 1, 1)

        def quad_body(t, carry):
            for u in range(4):
                i4 = t * 4 + u
                b2, b4 = u % 2, u

                @pl.when(i4 > 1)
                def _():
                    scatter_copy(b2, (u + 2) % 4).wait()
                wait_load(b2, b4)
                gcp = pltpu.make_async_copy(tab_hbm.at[idx_vs[b2]],
                                            rows_vs[b2], gsems[b2])
                gcp.start()
                gcp.wait()

                @pl.when(i4 + 2 < n_chunks)
                def _():
                    start_load(i4 + 2, b2, (u + 2) % 4)
                scatter_copy(b2, b4).start()
            return carry

        lax.fori_loop(0, n_quad, quad_body, 0)
        scatter_copy(0, 2).wait()
        scatter_copy(1, 3).wait()

    return gather_k


def _norm_body(L, D, g_ref, o_ref):
    z = g_ref[...]
    ss = jnp.sum(z * z, axis=0)
    inv = 1.0 / jnp.maximum(jnp.sqrt(ss), 1e-12)
    o = z * inv[None]
    for l in range(L):
        o_ref[l, :, :] = o[l, :, :D].T


def _norm_t(g3, L, D):
    B = g3.shape[1]
    return pl.pallas_call(
        functools.partial(_norm_body, L, D),
        grid=(B // _BS3,),
        in_specs=[pl.BlockSpec((L, _BS3, 2 * D), lambda i: (0, i, 0))],
        out_specs=pl.BlockSpec((L, D, _BS3), lambda i: (0, 0, i)),
        out_shape=jax.ShapeDtypeStruct((L, D, B), jnp.float32),
    )(g3)


def kernel(x, table, W1, b1, W2, b2):
    B, L = x.shape
    V, D = table.shape
    NIDX = B * L
    idx = x.reshape(NIDX)
    j = jnp.arange(NIDX, dtype=jnp.int32)
    dst = (j % L) * B + (j // L)

    packed = _mlp_pack(table, W1, b1.reshape(1, D), W2, b2.reshape(1, D))
    g2 = _make_sc_gather(2 * D, NIDX)(idx, dst, packed)
    g3 = g2.reshape(L, B, 2 * D)
    o = _norm_t(g3, L, D)
    return jnp.transpose(o, (2, 0, 1))


# speedup vs baseline: 1.0853x; 1.0853x over previous
import functools
import math

import jax
import jax.numpy as jnp
from jax import lax
from jax.experimental import pallas as pl
from jax.experimental.pallas import tpu as pltpu
from jax.experimental.pallas import tpu_sc as plsc

_NW = 32
_CHUNK = 400
_BS1 = 20000
_BS3 = 256


def _mlp_pack_body(t_ref, w1_ref, b1_ref, w2_ref, b2_ref, o_ref):
    t = t_ref[...]
    h = jnp.dot(t, w1_ref[...], preferred_element_type=jnp.float32) + b1_ref[...]
    h = 0.5 * h * (1.0 + lax.erf(h * (1.0 / math.sqrt(2.0))))
    h = jnp.dot(h, w2_ref[...], preferred_element_type=jnp.float32) + b2_ref[...]
    o_ref[...] = jnp.concatenate([h, h], axis=1)


def _mlp_pack(table, W1, b1, W2, b2):
    V, D = table.shape
    return pl.pallas_call(
        _mlp_pack_body,
        grid=(V // _BS1,),
        in_specs=[
            pl.BlockSpec((_BS1, D), lambda i: (i, 0)),
            pl.BlockSpec((D, D), lambda i: (0, 0)),
            pl.BlockSpec((1, D), lambda i: (0, 0)),
            pl.BlockSpec((D, D), lambda i: (0, 0)),
            pl.BlockSpec((1, D), lambda i: (0, 0)),
        ],
        out_specs=pl.BlockSpec((_BS1, 2 * D), lambda i: (i, 0)),
        out_shape=jax.ShapeDtypeStruct((V, 2 * D), jnp.float32),
    )(table, W1, b1, W2, b2)


def _make_sc_gather(D2, NIDX):
    b_per_w = NIDX // _NW
    n_chunks = b_per_w // _CHUNK
    n_super = n_chunks // 2
    mesh = plsc.VectorSubcoreMesh(core_axis_name="c", subcore_axis_name="s")

    n_quad = n_chunks // 4

    @functools.partial(
        pl.kernel,
        mesh=mesh,
        out_type=jax.ShapeDtypeStruct((NIDX, D2), jnp.float32),
        scratch_types=[
            [pltpu.VMEM((_CHUNK,), jnp.int32) for _ in range(2)],
            [pltpu.VMEM((_CHUNK,), jnp.int32) for _ in range(4)],
            [pltpu.VMEM((_CHUNK, D2), jnp.float32) for _ in range(2)],
            [pltpu.SemaphoreType.DMA for _ in range(4)],
            [pltpu.SemaphoreType.DMA for _ in range(4)],
            [pltpu.SemaphoreType.DMA for _ in range(2)],
            [pltpu.SemaphoreType.DMA for _ in range(2)],
        ],
    )
    def gather_k(idx_hbm, dst_hbm, tab_hbm, out_hbm,
                 idx_vs, dst_vs, rows_vs, lisems, ldsems, gsems, ssems):
        wid = lax.axis_index("s") * 2 + lax.axis_index("c")
        wbase = wid * b_per_w

        def start_load(i, b2, b4):
            base = wbase + i * _CHUNK
            pltpu.async_copy(idx_hbm.at[pl.ds(base, _CHUNK)],
                             idx_vs[b2], lisems[b4])
            pltpu.async_copy(dst_hbm.at[pl.ds(base, _CHUNK)],
                             dst_vs[b4], ldsems[b4])

        def wait_load(b2, b4):
            pltpu.make_async_copy(idx_hbm.at[pl.ds(wbase, _CHUNK)],
                                  idx_vs[b2], lisems[b4]).wait()
            pltpu.make_async_copy(dst_hbm.at[pl.ds(wbase, _CHUNK)],
                                  dst_vs[b4], ldsems[b4]).wait()

        def scatter_copy(b2, b4):
            return pltpu.make_async_copy(rows_vs[b2], out_hbm.at[dst_vs[b4]],
                                         ssems[b2])

        def gather_copy(b2):
            return pltpu.make_async_copy(tab_hbm.at[idx_vs[b2]],
                                         rows_vs[b2], gsems[b2])

        start_load(0, 0, 0)
        wait_load(0, 0)
        gather_copy(0).start()
        start_load(1, 1, 1)

        def quad_body(t, carry):
            for u in range(4):
                i4 = t * 4 + u
                b2, b4 = u % 2, u
                nb2, nb4 = (u + 1) % 2, (u + 1) % 4

                @pl.when(i4 + 1 < n_chunks)
                def _():
                    wait_load(nb2, nb4)

                    @pl.when(i4 > 0)
                    def _():
                        scatter_copy(nb2, (u + 3) % 4).wait()
                    gather_copy(nb2).start()
                gather_copy(b2).wait()

                @pl.when(i4 + 2 < n_chunks)
                def _():
                    start_load(i4 + 2, b2, (u + 2) % 4)
                scatter_copy(b2, b4).start()
            return carry

        lax.fori_loop(0, n_quad, quad_body, 0)
        scatter_copy(0, 2).wait()
        scatter_copy(1, 3).wait()

    return gather_k


def _norm_body(L, D, g_ref, o_ref):
    z = g_ref[...]
    ss = jnp.sum(z * z, axis=0)
    inv = 1.0 / jnp.maximum(jnp.sqrt(ss), 1e-12)
    o = z * inv[None]
    for l in range(L):
        o_ref[l, :, :] = o[l, :, :D].T


def _norm_t(g3, L, D):
    B = g3.shape[1]
    return pl.pallas_call(
        functools.partial(_norm_body, L, D),
        grid=(B // _BS3,),
        in_specs=[pl.BlockSpec((L, _BS3, 2 * D), lambda i: (0, i, 0))],
        out_specs=pl.BlockSpec((L, D, _BS3), lambda i: (0, 0, i)),
        out_shape=jax.ShapeDtypeStruct((L, D, B), jnp.float32),
    )(g3)


def kernel(x, table, W1, b1, W2, b2):
    B, L = x.shape
    V, D = table.shape
    NIDX = B * L
    idx = x.reshape(NIDX)
    j = jnp.arange(NIDX, dtype=jnp.int32)
    dst = (j % L) * B + (j // L)

    packed = _mlp_pack(table, W1, b1.reshape(1, D), W2, b2.reshape(1, D))
    g2 = _make_sc_gather(2 * D, NIDX)(idx, dst, packed)
    g3 = g2.reshape(L, B, 2 * D)
    o = _norm_t(g3, L, D)
    return jnp.transpose(o, (2, 0, 1))
